# baseline (speedup 1.0000x reference)
import jax
import jax.numpy as jnp
from jax import lax
from jax.experimental import pallas as pl
from jax.experimental.pallas import tpu as pltpu
from jax.experimental.pallas import tpu_sc as plsc

B, S, D = 4, 4096, 1024
NC, NS = 2, 16
NW = NC * NS
SPW = S // NW
RPC = 16
NCH = SPW // RPC
CHUNK = RPC * D


def _sc_body(emb_hbm, pos_hbm, out_hbm, p_buf, e_buf):
    wid = lax.axis_index("s") * NC + lax.axis_index("c")
    base = wid * (SPW * D)

    def chunk(i, carry):
        p_off = base + i * CHUNK
        pltpu.sync_copy(pos_hbm.at[pl.ds(p_off, CHUNK)], p_buf)

        def batch(b, carry2):
            e_off = b * (S * D) + p_off
            pltpu.sync_copy(emb_hbm.at[pl.ds(e_off, CHUNK)], e_buf)

            @plsc.parallel_loop(0, CHUNK, step=16, unroll=8)
            def add(j):
                plsc.addupdate(e_buf.at[pl.ds(j, 16)], p_buf[pl.ds(j, 16)])

            pltpu.sync_copy(e_buf, out_hbm.at[pl.ds(e_off, CHUNK)])
            return carry2

        return lax.fori_loop(0, B, batch, carry)

    lax.fori_loop(0, NCH, chunk, 0)


def kernel(embeddings, pos_table):
    b, s, d = embeddings.shape
    mesh = plsc.VectorSubcoreMesh(core_axis_name="c", subcore_axis_name="s")
    out = pl.kernel(
        _sc_body,
        out_type=jax.ShapeDtypeStruct((b * s * d,), embeddings.dtype),
        mesh=mesh,
        scratch_types=[
            pltpu.VMEM((CHUNK,), jnp.float32),
            pltpu.VMEM((CHUNK,), jnp.float32),
        ],
    )(embeddings.reshape(-1), pos_table[:s].reshape(-1))
    return out.reshape(b, s, d)


# speedup vs baseline: 1.2218x; 1.2218x over previous
import jax
import jax.numpy as jnp
from jax import lax
from jax.experimental import pallas as pl
from jax.experimental.pallas import tpu as pltpu
from jax.experimental.pallas import tpu_sc as plsc

B, S, D = 4, 4096, 1024
NC, NS = 2, 16
NW = NC * NS
SPW = S // NW
RPC = 32
NCH = SPW // RPC
CHUNK = RPC * D
NT = NCH * B


def _sc_body(emb_hbm, pos_hbm, out_hbm, p_buf, e_a, e_b,
             p_sem, i_sem_a, i_sem_b, o_sem_a, o_sem_b):
    wid = lax.axis_index("s") * NC + lax.axis_index("c")
    base = wid * (SPW * D)

    ebufs = (e_a, e_b)
    isems = (i_sem_a, i_sem_b)
    osems = (o_sem_a, o_sem_b)

    in_dma = [None] * NT
    out_dma = [None] * NT

    pos_dma = pltpu.async_copy(pos_hbm.at[pl.ds(base, CHUNK)], p_buf, p_sem)
    in_dma[0] = pltpu.async_copy(
        emb_hbm.at[pl.ds(base, CHUNK)], ebufs[0], isems[0])

    for t in range(NT):
        c, b = divmod(t, B)
        buf = t & 1
        if b == 0:
            pos_dma.wait()
        in_dma[t].wait()
        if t + 1 < NT:
            if t >= 1:
                out_dma[t - 1].wait()
            c1, b1 = divmod(t + 1, B)
            off1 = b1 * (S * D) + base + c1 * CHUNK
            in_dma[t + 1] = pltpu.async_copy(
                emb_hbm.at[pl.ds(off1, CHUNK)], ebufs[(t + 1) & 1],
                isems[(t + 1) & 1])

        e = ebufs[buf]

        @plsc.parallel_loop(0, CHUNK, step=16, unroll=8)
        def add(j, _e=e):
            plsc.addupdate(_e.at[pl.ds(j, 16)], p_buf[pl.ds(j, 16)])

        off = b * (S * D) + base + c * CHUNK
        out_dma[t] = pltpu.async_copy(e, out_hbm.at[pl.ds(off, CHUNK)],
                                      osems[buf])
        if b == B - 1 and c + 1 < NCH:
            pos_dma = pltpu.async_copy(
                pos_hbm.at[pl.ds(base + (c + 1) * CHUNK, CHUNK)], p_buf, p_sem)

    out_dma[NT - 2].wait()
    out_dma[NT - 1].wait()


def kernel(embeddings, pos_table):
    b, s, d = embeddings.shape
    mesh = plsc.VectorSubcoreMesh(core_axis_name="c", subcore_axis_name="s")
    out = pl.kernel(
        _sc_body,
        out_type=jax.ShapeDtypeStruct((b * s * d,), embeddings.dtype),
        mesh=mesh,
        scratch_types=[
            pltpu.VMEM((CHUNK,), jnp.float32),
            pltpu.VMEM((CHUNK,), jnp.float32),
            pltpu.VMEM((CHUNK,), jnp.float32),
            pltpu.SemaphoreType.DMA,
            pltpu.SemaphoreType.DMA,
            pltpu.SemaphoreType.DMA,
            pltpu.SemaphoreType.DMA,
            pltpu.SemaphoreType.DMA,
        ],
    )(embeddings.reshape(-1), pos_table[:s].reshape(-1))
    return out.reshape(b, s, d)


# speedup vs baseline: 3.1652x; 2.5906x over previous
import jax
import jax.numpy as jnp
from jax import lax
from jax.experimental import pallas as pl
from jax.experimental.pallas import tpu as pltpu
from jax.experimental.pallas import tpu_sc as plsc

B, S, D = 4, 4096, 1024
NC, NS = 2, 16
NW = NC * NS
SPW = S // NW
RPC = 32
NCH = SPW // RPC
CHUNK = RPC * D
NT = NCH * B


def _sc_body(emb_hbm, pos_hbm, out_hbm, p_buf, e_a, e_b,
             p_sem, i_sem_a, i_sem_b, o_sem_a, o_sem_b):
    wid = lax.axis_index("s") * NC + lax.axis_index("c")
    s_base = wid * SPW

    ebufs = (e_a, e_b)
    isems = (i_sem_a, i_sem_b)
    osems = (o_sem_a, o_sem_b)

    in_dma = [None] * NT
    out_dma = [None] * NT

    pos_dma = pltpu.async_copy(
        pos_hbm.at[pl.ds(s_base, RPC), :], p_buf, p_sem)
    in_dma[0] = pltpu.async_copy(
        emb_hbm.at[0, pl.ds(s_base, RPC), :], ebufs[0], isems[0])

    for t in range(NT):
        c, b = divmod(t, B)
        buf = t & 1
        if b == 0:
            pos_dma.wait()
        in_dma[t].wait()
        if t + 1 < NT:
            if t >= 1:
                out_dma[t - 1].wait()
            c1, b1 = divmod(t + 1, B)
            in_dma[t + 1] = pltpu.async_copy(
                emb_hbm.at[b1, pl.ds(s_base + c1 * RPC, RPC), :],
                ebufs[(t + 1) & 1], isems[(t + 1) & 1])

        e = ebufs[buf]

        @plsc.parallel_loop(0, CHUNK, step=16, unroll=8)
        def add(j, _e=e):
            r = lax.shift_right_logical(j, 10)
            col = pl.multiple_of(lax.bitwise_and(j, D - 1), 16)
            plsc.addupdate(_e.at[r, pl.ds(col, 16)],
                           p_buf[r, pl.ds(col, 16)])

        out_dma[t] = pltpu.async_copy(
            e, out_hbm.at[b, pl.ds(s_base + c * RPC, RPC), :], osems[buf])
        if b == B - 1 and c + 1 < NCH:
            pos_dma = pltpu.async_copy(
                pos_hbm.at[pl.ds(s_base + (c + 1) * RPC, RPC), :],
                p_buf, p_sem)

    out_dma[NT - 2].wait()
    out_dma[NT - 1].wait()


def kernel(embeddings, pos_table):
    b, s, d = embeddings.shape
    mesh = plsc.VectorSubcoreMesh(core_axis_name="c", subcore_axis_name="s")
    return pl.kernel(
        _sc_body,
        out_type=jax.ShapeDtypeStruct((b, s, d), embeddings.dtype),
        mesh=mesh,
        scratch_types=[
            pltpu.VMEM((RPC, D), jnp.float32),
            pltpu.VMEM((RPC, D), jnp.float32),
            pltpu.VMEM((RPC, D), jnp.float32),
            pltpu.SemaphoreType.DMA,
            pltpu.SemaphoreType.DMA,
            pltpu.SemaphoreType.DMA,
            pltpu.SemaphoreType.DMA,
            pltpu.SemaphoreType.DMA,
        ],
    )(embeddings, pos_table[:s])
